# Initial kernel scaffold; baseline (speedup 1.0000x reference)
import jax
import jax.numpy as jnp
from jax.experimental import pallas as pl


def kernel(fnode, fmess, agraph, bgraph, mask, params):
    raise NotImplementedError("write your pallas kernel here")


# speedup vs baseline: 8.2022x; 8.2022x over previous
import functools

import jax
import jax.numpy as jnp
import numpy as np
from jax import lax
from jax.experimental import pallas as pl
from jax.experimental.pallas import tpu as pltpu
from jax.experimental.pallas import tpu_sc as plsc

N_NODES = 10000
N_EDGES = 160000
MAX_NN = 4
H = 128
IN_SIZE = 128
HEADS = 4
DPH = H // HEADS

EDGE_BLK = 1600
NODE_BLK = 1000
N_NODES_PAD = 10240
SC_B = 64
NW = 32


def _lrelu(x):
    return jnp.where(x >= 0, x, 0.01 * x)


def _mmf(a, b):
    return jnp.dot(a, b, preferred_element_type=jnp.float32)


def _mm(a, b):
    return jnp.dot(a.astype(jnp.bfloat16), b.astype(jnp.bfloat16),
                   preferred_element_type=jnp.float32)


_MASKHI = -65536


def _pack_table(v, sk, sel_e, sel_o):
    f32, i32 = jnp.float32, jnp.int32
    ve = _mm(v, sel_e)
    vo = _mm(v, sel_o)
    be = lax.shift_right_logical(
        lax.bitcast_convert_type(ve.astype(jnp.bfloat16).astype(f32), i32), 16)
    bo = lax.bitcast_convert_type(
        vo.astype(jnp.bfloat16).astype(f32), i32) & _MASKHI
    words_v = be | bo
    mn = sk.astype(jnp.bfloat16).astype(f32)
    rs = sk - mn
    wm = lax.shift_right_logical(lax.bitcast_convert_type(mn, i32), 16)
    wr_ = lax.bitcast_convert_type(rs, i32) & _MASKHI
    words_s = wm | wr_
    pad = jnp.zeros((v.shape[0], 48), i32)
    return jnp.concatenate([words_v, words_s, pad], axis=1)


def _edge_pre_body2(x_ref, bg_ref, wzx, wr, whx, whh, wq, wv, wk, amq, amk,
                    sel_e, sel_o, ca, ab16, tp, sq):
    x = x_ref[...]
    cz = ca[0:1, :]
    cr = ca[1:2, :]
    bh = ca[2:3, :]
    bv = ca[3:4, :]
    bq = ca[4:5, :]
    bk = ca[5:6, :]
    z1 = jax.nn.sigmoid(_mm(x, wzx[...]) + cz)
    r1 = jax.nn.sigmoid(_mm(x, wr[...]) + cr)
    pre1 = jnp.tanh(_mm(x, whx[...]) + _mm(r1 * bv, whh[...]) + bh)
    i = pl.program_id(0)
    rows = lax.broadcasted_iota(jnp.int32, (x.shape[0], 1), 0)
    m = jnp.where((rows == 0) & (i == 0), 0.0, 1.0)
    h1 = ((1.0 - z1) * bv + z1 * pre1) * m
    v = _mm(h1, wv[...]) + bv
    k1 = _lrelu(_mm(h1, wk[...]) + bk)
    s = jnp.sum(h1, axis=1, keepdims=True)
    sk = jnp.where(s == 0.0, -1e30, _mm(k1, amk[...]))
    tp[...] = _pack_table(v, sk, sel_e[...], sel_o[...])
    n = x.shape[0]
    bgf = jnp.concatenate(
        [jnp.zeros((n, 4), jnp.float32), bg_ref[...].astype(jnp.float32),
         jnp.zeros((n, H - 8), jnp.float32)], axis=1)
    sq[...] = _mm(_lrelu(_mm(x, wq[...]) + bq), amq[...]) + ab16[...] + bgf


def _gru_enc_body(x_ref, sp_ref, pm, wzx, wzh, wr, ur, whx, whh, wke, wve,
                  amk2, sel_e, sel_o, cb, h2_ref, tp2, *, blk0=0):
    x = x_ref[...]
    s = _mmf(sp_ref[...], pm[...])
    bz = cb[0:1, :]
    bur = cb[1:2, :]
    bh = cb[2:3, :]
    bke = cb[3:4, :]
    bve = cb[4:5, :]
    z = jax.nn.sigmoid(_mm(x, wzx[...]) + _mm(s, wzh[...]) + bz)
    r = jax.nn.sigmoid(_mm(x, wr[...]) + _mm(s, ur[...]) + bur)
    pre = jnp.tanh(_mm(x, whx[...]) + _mm(r * s, whh[...]) + bh)
    i = pl.program_id(0)
    rows = lax.broadcasted_iota(jnp.int32, (x.shape[0], 1), 0)
    m = jnp.where((rows == 0) & (i + blk0 == 0), 0.0, 1.0)
    h2 = ((1.0 - z) * s + z * pre) * m
    h2_ref[...] = h2
    v2 = _mm(h2, wve[...]) + bve
    k2 = _lrelu(_mm(h2, wke[...]) + bke)
    s2 = jnp.sum(h2, axis=1, keepdims=True)
    sk2 = jnp.where(s2 == 0.0, -1e30, _mm(k2, amk2[...]))
    tp2[...] = _pack_table(v2, sk2, sel_e[...], sel_o[...])


def _node_pre_body(fn_ref, ag_ref, wqe, amq2, cb, sq2):
    fn = fn_ref[...]
    bqe = cb[0:1, :]
    n = fn.shape[0]
    agf = jnp.concatenate(
        [jnp.zeros((n, 4), jnp.float32), ag_ref[...].astype(jnp.float32),
         jnp.zeros((n, H - 8), jnp.float32)], axis=1)
    sq2[...] = (_mm(_lrelu(_mm(fn, wqe[...]) + bqe), amq2[...])
                + cb[1:2, :] + agf)


def _node_out_body(fn_ref, nm_ref, wox, woh, cb, mask_ref, out_ref):
    fn = fn_ref[...]
    nm = nm_ref[...]
    bo = cb[0:1, :]
    nh = _mm(fn, wox[...]) + _mm(nm, woh[...]) + bo
    nh = nh * 0.5 * (1.0 + lax.erf(nh * 0.7071067811865476))
    out_ref[...] = nh * mask_ref[...]


def _gru_enc_body_alias(x_ref, sp_ref, pm, wzx, wzh, wr, ur, whx, whh, wke,
                        wve, amk2, sel_e, sel_o, cb, h2d, tpd, h2_ref, tp2,
                        *, blk0=0):
    del h2d, tpd
    _gru_enc_body(x_ref, sp_ref, pm, wzx, wzh, wr, ur, whx, whh, wke, wve,
                  amk2, sel_e, sel_o, cb, h2_ref, tp2, blk0=blk0)


def _rep_spec(shape):
    nd = len(shape)
    return pl.BlockSpec(shape, lambda i: (0,) * nd)


def _row_spec(blk, w, off=0):
    return pl.BlockSpec((blk, w), lambda i: (i + off, 0))


def _vbcast(x, lane_idx):
    dn = lax.GatherDimensionNumbers(offset_dims=(), collapsed_slice_dims=(0,),
                                    start_index_map=(0,))
    idx = jnp.full((16,), lane_idx, jnp.int32)
    return lax.gather(x, idx[:, None], dn, (1,),
                      mode=lax.GatherScatterMode.PROMISE_IN_BOUNDS)


def _make_gather_attn(n_rows, row0=0):
    nb = n_rows // SC_B
    nr4 = SC_B * MAX_NN
    mesh = plsc.VectorSubcoreMesh(core_axis_name="c", subcore_axis_name="s")

    @functools.partial(
        pl.kernel, mesh=mesh,
        out_type=jax.ShapeDtypeStruct((n_rows, H), jnp.float32),
        compiler_params=pltpu.CompilerParams(needs_layout_passes=False),
        scratch_types=[
            pltpu.VMEM((2 * nr4,), jnp.int32),
            pltpu.VMEM((2 * nr4, H), jnp.int32),
            pltpu.VMEM((2 * SC_B, H), jnp.float32),
            pltpu.VMEM((SC_B, H), jnp.float32),
            pltpu.SemaphoreType.DMA,
            pltpu.SemaphoreType.DMA,
        ],
    )
    def k(tp_hbm, sq_hbm, out_hbm, idx_v, rows_v, sq_v, o_v,
          sem0, sem1):
        wid = lax.axis_index("s") * 2 + lax.axis_index("c")
        nk = (nb - wid + NW - 1) // NW
        sems = (sem0, sem1)
        lane = lax.iota(jnp.int32, 16)

        def gather_cps(s, make=False):
            f = pltpu.make_async_copy if make else pltpu.async_copy
            return [f(tp_hbm.at[idx_v.at[pl.ds(s * nr4 + c * 128, 128)]],
                      rows_v.at[pl.ds(s * nr4 + c * 128, 128)], sems[s])
                    for c in range(nr4 // 128)]

        def issue(kk, s):
            b0 = row0 + (wid + kk * NW) * SC_B
            pltpu.sync_copy(sq_hbm.at[pl.ds(b0, SC_B)],
                            sq_v.at[pl.ds(s * SC_B, SC_B)])
            for g in range(nr4 // 16):
                vals = plsc.load_gather(
                    sq_v, [s * SC_B + g * 4 + (lane >> 2), 4 + (lane & 3)])
                idx_v[pl.ds(s * nr4 + g * 16, 16)] = vals.astype(jnp.int32)
            gather_cps(s)

        def unpack(word):
            lo = plsc.bitcast(lax.shift_left(word, 16), jnp.float32)
            hi = plsc.bitcast(word & _MASKHI, jnp.float32)
            return lo, hi

        def compute(kk, s):
            row0 = s * nr4
            sq0 = s * SC_B

            @plsc.parallel_loop(0, SC_B, unroll=2)
            def edge_body(e):
                sqrow = sq_v[sq0 + e, pl.ds(0, 16)]
                sc = []
                for j in range(MAX_NN):
                    mn, rs = unpack(rows_v[row0 + e * MAX_NN + j,
                                           pl.ds(64, 16)])
                    sc.append(sqrow + mn + rs)
                mx = jnp.maximum(jnp.maximum(sc[0], sc[1]),
                                 jnp.maximum(sc[2], sc[3]))
                ex = [jnp.exp(sc[j] - mx) for j in range(MAX_NN)]
                inv = 1.0 / (ex[0] + ex[1] + ex[2] + ex[3])
                wv = [ex[j] * inv for j in range(MAX_NN)]
                for w in range(HEADS):
                    acc_lo = jnp.zeros((16,), jnp.float32)
                    acc_hi = jnp.zeros((16,), jnp.float32)
                    for j in range(MAX_NN):
                        wb = _vbcast(wv[j], w)
                        lo, hi = unpack(rows_v[row0 + e * MAX_NN + j,
                                               pl.ds(w * 16, 16)])
                        acc_lo = acc_lo + wb * lo
                        acc_hi = acc_hi + wb * hi
                    o_v[e, pl.ds(32 * w, 16)] = acc_lo
                    o_v[e, pl.ds(32 * w + 16, 16)] = acc_hi

            b0 = (wid + kk * NW) * SC_B
            pltpu.sync_copy(o_v, out_hbm.at[pl.ds(b0, SC_B)])

        @pl.when(nk > 0)
        def _():
            issue(0, 0)

        def pair_body(p, _):
            k0 = p * 2
            k1 = k0 + 1
            for cp in gather_cps(0, make=True):
                cp.wait()

            @pl.when(k1 < nk)
            def _():
                issue(k1, 1)

            compute(k0, 0)

            @pl.when(k1 < nk)
            def _():
                for cp in gather_cps(1, make=True):
                    cp.wait()

                @pl.when(k1 + 1 < nk)
                def _():
                    issue(k1 + 1, 0)

                compute(k1, 1)
            return 0

        lax.fori_loop(0, (nk + 1) // 2, pair_body, 0)

    return k


_gather_attn_cache = {}


def _gather_attn(n_rows, row0=0):
    key = (n_rows, row0)
    if key not in _gather_attn_cache:
        _gather_attn_cache[key] = _make_gather_attn(n_rows, row0)
    return _gather_attn_cache[key]


def _head_mat(w):
    e = np.zeros((H, w), np.float32)
    e[np.arange(H), np.arange(H) // DPH] = 1.0
    return e


_EQ128 = jnp.asarray(_head_mat(H))
_EK16 = jnp.asarray(_head_mat(16))
_SEL_E = jnp.asarray(
    np.eye(H, dtype=np.float32)[:, 0::2])
_SEL_O = jnp.asarray(np.eye(H, dtype=np.float32)[:, 1::2])


def _perm_mat():
    p = np.arange(H)
    w, t = p // 32, p % 32
    d = 32 * w + np.where(t < 16, 2 * t, 2 * (t - 16) + 1)
    pmat = np.zeros((H, H), np.float32)
    pmat[p, d] = 1.0
    return pmat


_PM = jnp.asarray(_perm_mat())


def _alpha_mat(alpha_half, w=16):
    e = _EQ128 if w == H else _EK16
    return alpha_half.reshape(-1)[:, None] * e


def kernel(fnode, fmess, agraph, bgraph, mask, params):
    pr = params["rnn"]
    pe = params["enc"]
    f32 = jnp.float32

    wzx, wzh = pr["W_z"][:IN_SIZE], pr["W_z"][IN_SIZE:]
    whx, whh = pr["W_h"][:IN_SIZE], pr["W_h"][IN_SIZE:]
    bv = pr["bv"]
    cz = pr["b_z"] + bv @ wzh
    cr = pr["b_ur"] + bv @ pr["U_r"]
    amq = _alpha_mat(pr["alpha"][0, 0, :, :DPH], H)
    amk = _alpha_mat(pr["alpha"][0, 0, :, DPH:])
    ab16 = jnp.concatenate([pr["abias"][0, 0],
                            jnp.zeros((H - 4,), f32)])[None, :]
    ca = jnp.stack([cz, cr, pr["b_h"], bv, pr["bq"], pr["bk"],
                    jnp.zeros((H,), f32), jnp.zeros((H,), f32)])
    sel_e, sel_o, pm = _SEL_E, _SEL_O, _PM

    n_eblk = N_EDGES // EDGE_BLK
    tp1, sq1 = pl.pallas_call(
        _edge_pre_body2,
        grid=(n_eblk,),
        in_specs=[
            _row_spec(EDGE_BLK, H), _row_spec(EDGE_BLK, MAX_NN),
            _rep_spec((H, H)), _rep_spec((H, H)), _rep_spec((H, H)),
            _rep_spec((H, H)), _rep_spec((H, H)), _rep_spec((H, H)),
            _rep_spec((H, H)),
            _rep_spec((H, H)), _rep_spec((H, 16)),
            _rep_spec((H, 64)), _rep_spec((H, 64)),
            _rep_spec((8, H)), _rep_spec((1, H)),
        ],
        out_specs=[
            _row_spec(EDGE_BLK, H),
            _row_spec(EDGE_BLK, H),
        ],
        out_shape=[
            jax.ShapeDtypeStruct((N_EDGES, H), jnp.int32),
            jax.ShapeDtypeStruct((N_EDGES, H), f32),
        ],
    )(fmess, bgraph.astype(jnp.int32), wzx, pr["W_r"], whx, whh,
      pr["Wq"], pr["Wv"], pr["Wk"], amq, amk, sel_e, sel_o, ca, ab16)

    half = N_EDGES // 2
    s_ctx_a = _gather_attn(half, 0)(tp1, sq1)
    s_ctx_b = _gather_attn(half, half)(tp1, sq1)

    amk2 = _alpha_mat(pe["alpha"][0, 0, :, DPH:])
    cb = jnp.stack([pr["b_z"], pr["b_ur"], pr["b_h"], pe["bk"], pe["bv"],
                    jnp.zeros((H,), f32), jnp.zeros((H,), f32),
                    jnp.zeros((H,), f32)])
    n_hblk = n_eblk // 2
    wcol = [_rep_spec((H, H))] * 9 + [_rep_spec((H, 16)),
                                      _rep_spec((H, 64)), _rep_spec((H, 64)),
                                      _rep_spec((8, H))]
    wargs = (pm, wzx, wzh, pr["W_r"], pr["U_r"], whx, whh,
             pe["Wk"], pe["Wv"], amk2, sel_e, sel_o, cb)
    h2a, tp2a = pl.pallas_call(
        _gru_enc_body,
        grid=(n_hblk,),
        in_specs=[_row_spec(EDGE_BLK, H), _row_spec(EDGE_BLK, H)] + wcol,
        out_specs=[
            _row_spec(EDGE_BLK, H),
            _row_spec(EDGE_BLK, H),
        ],
        out_shape=[
            jax.ShapeDtypeStruct((N_EDGES, H), f32),
            jax.ShapeDtypeStruct((N_EDGES, H), jnp.int32),
        ],
    )(fmess, s_ctx_a, *wargs)
    h2, tp2 = pl.pallas_call(
        functools.partial(_gru_enc_body_alias, blk0=n_hblk),
        grid=(n_hblk,),
        in_specs=([_row_spec(EDGE_BLK, H, n_hblk), _row_spec(EDGE_BLK, H)]
                  + wcol
                  + [pl.BlockSpec(memory_space=pl.ANY),
                     pl.BlockSpec(memory_space=pl.ANY)]),
        out_specs=[
            _row_spec(EDGE_BLK, H, n_hblk),
            _row_spec(EDGE_BLK, H, n_hblk),
        ],
        out_shape=[
            jax.ShapeDtypeStruct((N_EDGES, H), f32),
            jax.ShapeDtypeStruct((N_EDGES, H), jnp.int32),
        ],
        input_output_aliases={15: 0, 16: 1},
    )(fmess, s_ctx_b, *wargs, h2a, tp2a)

    amq2 = _alpha_mat(pe["alpha"][0, 0, :, :DPH], H)
    ab2row = jnp.concatenate([pe["abias"][0, 0],
                              jnp.zeros((H - 4,), f32)])
    cb2 = jnp.stack([pe["bq"], ab2row] + [jnp.zeros((H,), f32)] * 6)
    n_nblk = N_NODES // NODE_BLK
    sq2 = pl.pallas_call(
        _node_pre_body,
        grid=(n_nblk,),
        in_specs=[
            _row_spec(NODE_BLK, H), _row_spec(NODE_BLK, MAX_NN),
            _rep_spec((H, H)), _rep_spec((H, H)), _rep_spec((8, H)),
        ],
        out_specs=[_row_spec(NODE_BLK, H)],
        out_shape=[jax.ShapeDtypeStruct((N_NODES, H), f32)],
    )(fnode, agraph.astype(jnp.int32), pe["Wq"], amq2, cb2)[0]

    sq2_pad = jnp.concatenate(
        [sq2, jnp.zeros((N_NODES_PAD - N_NODES, H), f32)])

    nm_pad = _gather_attn(N_NODES_PAD)(tp2, sq2_pad)
    nm = nm_pad[:N_NODES]

    wox, woh = pe["Wo"][:H], pm @ pe["Wo"][H:]
    cb3 = jnp.stack([pe["bo"]] + [jnp.zeros((H,), f32)] * 7)
    nh = pl.pallas_call(
        _node_out_body,
        grid=(n_nblk,),
        in_specs=[
            _row_spec(NODE_BLK, H), _row_spec(NODE_BLK, H),
            _rep_spec((H, H)), _rep_spec((H, H)), _rep_spec((8, H)),
            _row_spec(NODE_BLK, 1),
        ],
        out_specs=[_row_spec(NODE_BLK, H)],
        out_shape=[jax.ShapeDtypeStruct((N_NODES, H), f32)],
    )(fnode, nm, wox, woh, cb3, mask)[0]

    return (nh, h2)
